# baseline (speedup 1.0000x reference)
import functools

import jax
import jax.numpy as jnp
from jax import lax
from jax.experimental import pallas as pl
from jax.experimental.pallas import tpu as pltpu
from jax.experimental.pallas import tpu_sc as plsc

_CHUNK = 256
_NBUF = 4


@functools.lru_cache(maxsize=None)
def _make_gather(B, D):
    info = plsc.get_sparse_core_info()
    nc, ns = info.num_cores, info.num_subcores
    nw = nc * ns
    assert B % nw == 0
    b_per_w = B // nw
    c = _CHUNK
    nb = _NBUF
    assert b_per_w % (c * nb) == 0
    n_chunks = b_per_w // c
    n_groups = n_chunks // nb
    mesh = plsc.VectorSubcoreMesh(core_axis_name="c", subcore_axis_name="s")

    @functools.partial(
        pl.kernel,
        mesh=mesh,
        compiler_params=pltpu.CompilerParams(use_tc_tiling_on_sc=False),
        out_type=jax.ShapeDtypeStruct((B, D), jnp.float32),
        scratch_types=[
            pltpu.VMEM((b_per_w,), jnp.int32),
            *[pltpu.VMEM((c, D), jnp.float32) for _ in range(nb)],
            *[pltpu.SemaphoreType.DMA for _ in range(2 * nb)],
        ],
    )
    def gather_kernel(table_hbm, idx_hbm, out_hbm, idx_all, *bufs_and_sems):
        rows = bufs_and_sems[:nb]
        gsem = bufs_and_sems[nb:2 * nb]
        ssem = bufs_and_sems[2 * nb:3 * nb]
        wid = lax.axis_index("s") * nc + lax.axis_index("c")
        base = wid * b_per_w
        pltpu.sync_copy(idx_hbm.at[pl.ds(base, b_per_w)], idx_all)

        def gather_desc(i, b):
            return pltpu.make_async_copy(
                table_hbm.at[idx_all.at[pl.ds(i * c, c)]], rows[b], gsem[b])

        def store_desc(i, b):
            return pltpu.make_async_copy(
                rows[b], out_hbm.at[pl.ds(base + i * c, c)], ssem[b])

        for b in range(nb - 1):
            gather_desc(b, b).start()

        def group(g, carry):
            i0 = g * nb
            for b in range(nb):
                i = i0 + b
                bb = (b - 1) % nb
                j = i + nb - 1

                @pl.when(j < n_chunks)
                def _():
                    @pl.when(i >= 1)
                    def _():
                        store_desc(i - 1, bb).wait()
                    gather_desc(j, bb).start()

                gather_desc(i, b).wait()
                store_desc(i, b).start()
            return carry

        lax.fori_loop(0, n_groups, group, 0)

        for k in range(nb):
            i = n_chunks - nb + k
            store_desc(i, i % nb).wait()

    return gather_kernel


def kernel(val_tok, embedding_weight):
    b, s = val_tok.shape
    d = embedding_weight.shape[1]
    idx = val_tok.reshape(-1).astype(jnp.int32)
    out = _make_gather(b * s, d)(embedding_weight, idx)
    return out.reshape(b, s, d)


# speedup vs baseline: 1.0009x; 1.0009x over previous
import functools

import jax
import jax.numpy as jnp
from jax import lax
from jax.experimental import pallas as pl
from jax.experimental.pallas import tpu as pltpu
from jax.experimental.pallas import tpu_sc as plsc

_CHUNK = 512
_NBUF = 2


@functools.lru_cache(maxsize=None)
def _make_gather(B, D):
    info = plsc.get_sparse_core_info()
    nc, ns = info.num_cores, info.num_subcores
    nw = nc * ns
    assert B % nw == 0
    b_per_w = B // nw
    c = _CHUNK
    nb = _NBUF
    assert b_per_w % (c * nb) == 0
    n_chunks = b_per_w // c
    n_groups = n_chunks // nb
    mesh = plsc.VectorSubcoreMesh(core_axis_name="c", subcore_axis_name="s")

    @functools.partial(
        pl.kernel,
        mesh=mesh,
        compiler_params=pltpu.CompilerParams(use_tc_tiling_on_sc=False),
        out_type=jax.ShapeDtypeStruct((B, D), jnp.float32),
        scratch_types=[
            pltpu.VMEM((b_per_w,), jnp.int32),
            *[pltpu.VMEM((c, D), jnp.float32) for _ in range(nb)],
            *[pltpu.SemaphoreType.DMA for _ in range(2 * nb)],
        ],
    )
    def gather_kernel(table_hbm, idx_hbm, out_hbm, idx_all, *bufs_and_sems):
        rows = bufs_and_sems[:nb]
        gsem = bufs_and_sems[nb:2 * nb]
        ssem = bufs_and_sems[2 * nb:3 * nb]
        wid = lax.axis_index("s") * nc + lax.axis_index("c")
        base = wid * b_per_w
        pltpu.sync_copy(idx_hbm.at[pl.ds(base, b_per_w)], idx_all)

        def gather_desc(i, b):
            return pltpu.make_async_copy(
                table_hbm.at[idx_all.at[pl.ds(i * c, c)]], rows[b], gsem[b])

        def store_desc(i, b):
            return pltpu.make_async_copy(
                rows[b], out_hbm.at[pl.ds(base + i * c, c)], ssem[b])

        for b in range(nb - 1):
            gather_desc(b, b).start()

        def group(g, carry):
            i0 = g * nb
            for b in range(nb):
                i = i0 + b
                bb = (b - 1) % nb
                j = i + nb - 1

                @pl.when(j < n_chunks)
                def _():
                    @pl.when(i >= 1)
                    def _():
                        store_desc(i - 1, bb).wait()
                    gather_desc(j, bb).start()

                gather_desc(i, b).wait()
                store_desc(i, b).start()
            return carry

        lax.fori_loop(0, n_groups, group, 0)

        for k in range(nb):
            i = n_chunks - nb + k
            store_desc(i, i % nb).wait()

    return gather_kernel


def kernel(val_tok, embedding_weight):
    b, s = val_tok.shape
    d = embedding_weight.shape[1]
    idx = val_tok.reshape(-1).astype(jnp.int32)
    out = _make_gather(b * s, d)(embedding_weight, idx)
    return out.reshape(b, s, d)
